# Initial kernel scaffold; baseline (speedup 1.0000x reference)
import jax
import jax.numpy as jnp
from jax.experimental import pallas as pl


def kernel(ids, embs, pad):
    raise NotImplementedError("write your pallas kernel here")


# speedup vs baseline: 1.2372x; 1.2372x over previous
import functools

import jax
import jax.numpy as jnp
from jax import lax
from jax.experimental import pallas as pl
from jax.experimental.pallas import tpu as pltpu
from jax.experimental.pallas import tpu_sc as plsc

EMBED_DIM = 32
CHUNK = 128


def _make_gather(num_workers, num_chunks):
  mesh = plsc.VectorSubcoreMesh(core_axis_name="c", subcore_axis_name="s")
  nc = mesh.num_cores

  @functools.partial(
      pl.kernel,
      out_type=jax.ShapeDtypeStruct(
          (num_workers, num_chunks, CHUNK, EMBED_DIM), jnp.float32
      ),
      mesh=mesh,
      scratch_types=[
          pltpu.VMEM((num_chunks, CHUNK), jnp.int32),
          pltpu.VMEM((CHUNK, EMBED_DIM), jnp.float32),
          pltpu.SemaphoreType.DMA,
      ],
      compiler_params=pltpu.CompilerParams(use_tc_tiling_on_sc=False),
  )
  def gather_kernel(table_hbm, ids_hbm, out_hbm, idx_v, rows_v, sem):
    wid = lax.axis_index("s") * nc + lax.axis_index("c")
    pltpu.sync_copy(ids_hbm.at[wid], idx_v)

    def step(j, carry):
      pltpu.async_copy(table_hbm.at[idx_v.at[j]], rows_v, sem).wait()
      pltpu.sync_copy(rows_v, out_hbm.at[wid].at[j])
      return carry

    lax.fori_loop(0, num_chunks, step, 0)

  return gather_kernel


def kernel(ids, embs, pad):
  del pad
  batch, hist = ids.shape
  total = batch * hist
  num_workers = 32
  per_worker = total // num_workers
  num_chunks = per_worker // CHUNK
  ids_r = ids.astype(jnp.int32).reshape(num_workers, num_chunks, CHUNK)
  out = _make_gather(num_workers, num_chunks)(embs, ids_r)
  return out.reshape(batch, hist, EMBED_DIM)


# speedup vs baseline: 1.3373x; 1.0809x over previous
import functools

import jax
import jax.numpy as jnp
from jax import lax
from jax.experimental import pallas as pl
from jax.experimental.pallas import tpu as pltpu
from jax.experimental.pallas import tpu_sc as plsc

EMBED_DIM = 32
CHUNK = 128
K = 10
SROWS = K * CHUNK


def _make_gather(num_workers, num_stages):
  mesh = plsc.VectorSubcoreMesh(core_axis_name="c", subcore_axis_name="s")
  nc = mesh.num_cores

  @functools.partial(
      pl.kernel,
      out_type=jax.ShapeDtypeStruct(
          (num_workers, num_stages, SROWS, EMBED_DIM), jnp.float32
      ),
      mesh=mesh,
      scratch_types=[
          pltpu.VMEM((num_stages, K, CHUNK), jnp.int32),
          pltpu.VMEM((SROWS, EMBED_DIM), jnp.float32),
          pltpu.VMEM((SROWS, EMBED_DIM), jnp.float32),
          pltpu.SemaphoreType.DMA,
          pltpu.SemaphoreType.DMA,
          pltpu.SemaphoreType.DMA,
          pltpu.SemaphoreType.DMA,
      ],
      compiler_params=pltpu.CompilerParams(use_tc_tiling_on_sc=False),
  )
  def gather_kernel(table_hbm, ids_hbm, out_hbm, idx_v, st0, st1, g0, g1,
                    w0, w1):
    wid = lax.axis_index("s") * nc + lax.axis_index("c")
    my_out = out_hbm.at[wid]
    pltpu.sync_copy(ids_hbm.at[wid], idx_v)

    def fire(g, stage, gsem):
      for k in range(K):
        pltpu.async_copy(
            table_hbm.at[idx_v.at[g].at[k]],
            stage.at[pl.ds(k * CHUNK, CHUNK)],
            gsem,
        )

    def drain_gathers(stage, gsem):
      pltpu.make_async_copy(my_out.at[0], stage, gsem).wait()

    fire(0, st0, g0)
    fire(1, st1, g1)

    def body(t, carry):
      g = 2 * t
      drain_gathers(st0, g0)
      wb0 = pltpu.async_copy(st0, my_out.at[g], w0)
      drain_gathers(st1, g1)
      wb1 = pltpu.async_copy(st1, my_out.at[g + 1], w1)
      wb0.wait()
      fire(g + 2, st0, g0)
      wb1.wait()
      fire(g + 3, st1, g1)
      return carry

    lax.fori_loop(0, num_stages // 2 - 1, body, 0)

    last = num_stages - 2
    drain_gathers(st0, g0)
    wb0 = pltpu.async_copy(st0, my_out.at[last], w0)
    drain_gathers(st1, g1)
    wb1 = pltpu.async_copy(st1, my_out.at[last + 1], w1)
    wb0.wait()
    wb1.wait()

  return gather_kernel


def kernel(ids, embs, pad):
  del pad
  batch, hist = ids.shape
  total = batch * hist
  num_workers = 32
  per_worker = total // num_workers
  num_stages = per_worker // SROWS
  ids_r = ids.astype(jnp.int32).reshape(num_workers, num_stages, K, CHUNK)
  out = _make_gather(num_workers, num_stages)(embs, ids_r)
  return out.reshape(batch, hist, EMBED_DIM)


# speedup vs baseline: 1.7106x; 1.2792x over previous
import functools

import jax
import jax.numpy as jnp
from jax import lax
from jax.experimental import pallas as pl
from jax.experimental.pallas import tpu as pltpu
from jax.experimental.pallas import tpu_sc as plsc

EMBED_DIM = 32
CHUNK = 128
HIST = 50
BLOCKS_PER_WORKER = 4
NT = HIST * BLOCKS_PER_WORKER


def _make_gather(vocab):
  mesh = plsc.VectorSubcoreMesh(core_axis_name="c", subcore_axis_name="s")
  nc = mesh.num_cores

  @functools.partial(
      pl.kernel,
      out_type=jax.ShapeDtypeStruct(
          (HIST, EMBED_DIM // 8, CHUNK, 8, CHUNK), jnp.float32
      ),
      mesh=mesh,
      scratch_types=[
          pltpu.VMEM((HIST, BLOCKS_PER_WORKER, CHUNK), jnp.int32),
          pltpu.VMEM((CHUNK, EMBED_DIM), jnp.float32),
          pltpu.VMEM((CHUNK, EMBED_DIM), jnp.float32),
          pltpu.VMEM((EMBED_DIM // 8, 8, CHUNK), jnp.float32),
          pltpu.VMEM((EMBED_DIM // 8, 8, CHUNK), jnp.float32),
          pltpu.SemaphoreType.DMA,
          pltpu.SemaphoreType.DMA,
          pltpu.SemaphoreType.DMA,
          pltpu.SemaphoreType.DMA,
      ],
      compiler_params=pltpu.CompilerParams(
          use_tc_tiling_on_sc=False, needs_layout_passes=False
      ),
  )
  def gather_kernel(table_hbm, ids_hbm, out_hbm, ids_v, rb0, rb1, tb0, tb1,
                    g0, g1, w0, w1):
    wid = lax.axis_index("s") * nc + lax.axis_index("c")

    pltpu.sync_copy(
        ids_hbm.at[:, pl.ds(wid * BLOCKS_PER_WORKER, BLOCKS_PER_WORKER)],
        ids_v,
    )

    lanes = lax.iota(jnp.int32, 16)

    def fire_gather(t, rb, sem):
      h = t // BLOCKS_PER_WORKER
      bl = t % BLOCKS_PER_WORKER
      pltpu.async_copy(table_hbm.at[ids_v.at[h, bl]], rb, sem)

    def wait_gather(rb, sem):
      pltpu.make_async_copy(table_hbm.at[pl.ds(0, CHUNK)], rb, sem).wait()

    def fire_write(t, tb, sem):
      h = t // BLOCKS_PER_WORKER
      bl = t % BLOCKS_PER_WORKER
      pltpu.async_copy(
          tb, out_hbm.at[h, :, wid * BLOCKS_PER_WORKER + bl], sem
      )

    def wait_write(tb, sem):
      pltpu.make_async_copy(tb, out_hbm.at[0, :, 0], sem).wait()

    def transpose(rb, tb):
      def block(i, carry):
        rows = i * 16 + lanes
        for d in range(EMBED_DIM):
          v = plsc.load_gather(rb, [rows, jnp.full((16,), d, jnp.int32)])
          tb[d // 8, d % 8, pl.ds(i * 16, 16)] = v
        return carry

      lax.fori_loop(0, CHUNK // 16, block, 0)

    fire_gather(0, rb0, g0)
    fire_gather(1, rb1, g1)

    wait_gather(rb0, g0)
    transpose(rb0, tb0)
    fire_gather(2, rb0, g0)
    plsc.subcore_barrier()
    fire_write(0, tb0, w0)
    wait_gather(rb1, g1)
    transpose(rb1, tb1)
    fire_gather(3, rb1, g1)
    plsc.subcore_barrier()
    fire_write(1, tb1, w1)

    def body(u, carry):
      t0 = 2 * u

      wait_gather(rb0, g0)
      wait_write(tb0, w0)
      transpose(rb0, tb0)

      @pl.when(t0 + 2 < NT)
      def _():
        fire_gather(t0 + 2, rb0, g0)

      plsc.subcore_barrier()
      fire_write(t0, tb0, w0)

      wait_gather(rb1, g1)
      wait_write(tb1, w1)
      transpose(rb1, tb1)

      @pl.when(t0 + 3 < NT)
      def _():
        fire_gather(t0 + 3, rb1, g1)

      plsc.subcore_barrier()
      fire_write(t0 + 1, tb1, w1)
      return carry

    lax.fori_loop(1, NT // 2, body, 0)

    wait_write(tb0, w0)
    wait_write(tb1, w1)

  return gather_kernel


def kernel(ids, embs, pad):
  del pad
  batch, hist = ids.shape
  ids_t = ids.astype(jnp.int32).T.reshape(hist, batch // CHUNK, CHUNK)
  out5 = _make_gather(embs.shape[0])(embs, ids_t)
  return out5.transpose(2, 4, 0, 1, 3).reshape(batch, hist, EMBED_DIM)
